# baseline (speedup 1.0000x reference)
import jax
import jax.numpy as jnp
from jax.experimental import pallas as pl
from jax.experimental.pallas import tpu as pltpu

R0 = 512
NBUF = 4


def _gate_body(x_hbm, wt_ref, b_ref, gate_ref, mask_ref, buf, sems):
    n_rows = x_hbm.shape[0]
    nch = n_rows // R0
    nf = wt_ref.shape[0] // R0

    def start(c, slot):
        pltpu.make_async_copy(
            x_hbm.at[pl.ds(c * R0, R0), :], buf.at[slot], sems.at[slot]
        ).start()

    for i in range(min(NBUF, nch)):
        start(i, i)

    h = jnp.broadcast_to(b_ref[...][None, :], (4, 16))
    hps = [[] for _ in range(4)]
    for c in range(nch):
        slot = c % NBUF
        pltpu.make_async_copy(
            x_hbm.at[pl.ds(c * R0, R0), :], buf.at[slot], sems.at[slot]
        ).wait()
        xb = buf[slot]
        s = (jnp.sum(xb, axis=-1) * (1.0 / 2048.0) + jnp.max(xb, axis=-1))[None, :]
        if c + NBUF < nch:
            start(c + NBUF, slot)
        wt = wt_ref[pl.ds((c % nf) * R0, R0), :]
        hp = jax.lax.dot_general(
            s, wt, (((1,), (0,)), ((), ())),
            preferred_element_type=jnp.float32,
        )
        hps[c // nf].append(hp)
    h = h + jnp.concatenate(
        [sum(parts[1:], parts[0]) for parts in hps], axis=0
    )

    h = jnp.where(h >= 0.0, h, 0.2 * h)
    iota = jax.lax.broadcasted_iota(jnp.int32, h.shape, 1)
    m1 = jnp.max(h, axis=1, keepdims=True)
    i1 = jnp.min(jnp.where(h == m1, iota, 16), axis=1, keepdims=True)
    h2 = jnp.where(iota == i1, -jnp.inf, h)
    m2 = jnp.max(h2, axis=1, keepdims=True)
    i2 = jnp.min(jnp.where(h2 == m2, iota, 16), axis=1, keepdims=True)
    sel = (iota == i1) | (iota == i2)
    mask_ref[...] = sel.astype(jnp.float32)
    d = jnp.where(sel, jnp.exp(h - m1), 0.0)
    gate_ref[...] = d / jnp.sum(d, axis=1, keepdims=True)


def kernel(x, W, b):
    B, F, C = x.shape
    E = W.shape[0]
    x2 = x.reshape(B * F, C)
    gating, mask = pl.pallas_call(
        _gate_body,
        in_specs=[
            pl.BlockSpec(memory_space=pl.ANY),
            pl.BlockSpec(memory_space=pltpu.VMEM),
            pl.BlockSpec(memory_space=pltpu.VMEM),
        ],
        out_specs=[
            pl.BlockSpec(memory_space=pltpu.VMEM),
            pl.BlockSpec(memory_space=pltpu.VMEM),
        ],
        out_shape=[
            jax.ShapeDtypeStruct((B, E), jnp.float32),
            jax.ShapeDtypeStruct((B, E), jnp.float32),
        ],
        scratch_shapes=[
            pltpu.VMEM((NBUF, R0, C), jnp.float32),
            pltpu.SemaphoreType.DMA((NBUF,)),
        ],
    )(x2, W.T, b)
    return gating, mask
